# baseline (speedup 1.0000x reference)
import functools

import jax
import jax.numpy as jnp
from jax import lax
from jax.experimental import pallas as pl
from jax.experimental.pallas import tpu as pltpu
from jax.experimental.pallas import tpu_sc as plsc

VOCAB = 8192
D = 8192
B = 16384
B_SC = 12288
B_TC = B - B_SC
FRAC = 4
PD = D // FRAC
NC = 2
NS = 16
NW = NC * NS
BPW = B_SC // NW
CHUNK = 8
NCH = BPW // CHUNK
NU = NCH * FRAC
NBUF = 4
TCK = 8


def _sc_body(idx_hbm, table_hbm, out_hbm, idx_v, rows_v, gsems, ssems):
    wid = lax.axis_index("s") * NC + lax.axis_index("c")
    base = wid * BPW
    pltpu.sync_copy(idx_hbm.at[wid], idx_v)

    def gather(u, b):
        g, h = u // FRAC, u % FRAC
        return pltpu.make_async_copy(
            table_hbm.at[idx_v.at[g], pl.ds(h * PD, PD)],
            rows_v.at[b], gsems.at[b])

    def scatter(u, b):
        g, h = u // FRAC, u % FRAC
        return pltpu.make_async_copy(
            rows_v.at[b],
            out_hbm.at[pl.ds(base + g * CHUNK, CHUNK), pl.ds(h * PD, PD)],
            ssems.at[b])

    for b in range(NBUF):
        gather(b, b).start()

    def step(u, b):
        bp = (b - 1) % NBUF
        gather(u, b).wait()
        scatter(u, b).start()
        scatter(u - 1, bp).wait()
        nxt = u - 1 + NBUF

        @pl.when(nxt < NU)
        def _():
            gather(nxt, bp).start()

    gather(0, 0).wait()
    scatter(0, 0).start()

    def body(t, carry):
        for b in range(NBUF):
            step(t * NBUF + b + 1, (b + 1) % NBUF)
        return carry

    lax.fori_loop(0, (NU - 1) // NBUF, body, 0)
    for i in range(NU - 1 - ((NU - 1) // NBUF) * NBUF):
        step((NU - 1) // NBUF * NBUF + 1 + i, (i + 1) % NBUF)
    scatter(NU - 1, (NU - 1) % NBUF).wait()


def _tc_body(idx_smem, table_hbm, out_hbm, sems):
    def copy(i, s):
        return pltpu.make_async_copy(
            table_hbm.at[pl.ds(idx_smem[i], 1)],
            out_hbm.at[pl.ds(i, 1)], sems.at[s])

    for s in range(TCK):
        copy(s, s).start()

    def body(t, carry):
        for s in range(TCK):
            i = t * TCK + s
            copy(i, s).wait()
            nxt = i + TCK

            @pl.when(nxt < B_TC)
            def _():
                copy(nxt, s).start()

        return carry

    lax.fori_loop(0, B_TC // TCK, body, 0)


@jax.jit
def _hybrid(idx_sc, idx_tc, table):
    mesh = plsc.VectorSubcoreMesh(core_axis_name="c", subcore_axis_name="s")
    sck = functools.partial(
        pl.kernel,
        mesh=mesh,
        out_type=jax.ShapeDtypeStruct((B_SC, D), jnp.float32),
        scratch_types=[
            pltpu.VMEM((NCH, CHUNK), jnp.int32),
            pltpu.VMEM((NBUF, CHUNK, PD), jnp.float32),
            pltpu.SemaphoreType.DMA((NBUF,)),
            pltpu.SemaphoreType.DMA((NBUF,)),
        ],
    )(_sc_body)
    out_sc = sck(idx_sc, table)
    out_tc = pl.pallas_call(
        _tc_body,
        out_shape=jax.ShapeDtypeStruct((B_TC, D), jnp.float32),
        in_specs=[
            pl.BlockSpec(memory_space=pltpu.SMEM),
            pl.BlockSpec(memory_space=pl.ANY),
        ],
        out_specs=pl.BlockSpec(memory_space=pl.ANY),
        scratch_shapes=[pltpu.SemaphoreType.DMA((TCK,))],
    )(idx_tc, table)
    return jnp.concatenate([out_sc, out_tc], axis=0)


def kernel(idx, table):
    idx32 = jnp.reshape(idx.astype(jnp.int32), (B,))
    idx_sc = jnp.reshape(idx32[:B_SC], (NW, NCH, CHUNK))
    idx_tc = idx32[B_SC:]
    return _hybrid(idx_sc, idx_tc, table)


# speedup vs baseline: 11.1597x; 11.1597x over previous
import functools

import jax
import jax.numpy as jnp
from jax import lax
from jax.experimental import pallas as pl
from jax.experimental.pallas import tpu as pltpu
from jax.experimental.pallas import tpu_sc as plsc

VOCAB = 8192
D = 8192
B = 16384
NC = 2
NS = 16
NW = NC * NS
BPW = B // NW
CHUNK = 4
NU = BPW // CHUNK
NBUF = 3


def _gather_body(idx_hbm, table_hbm, out_hbm, idx_v, rows_v, gsems, ssems):
    wid = lax.axis_index("s") * NC + lax.axis_index("c")
    base = wid * BPW
    pltpu.sync_copy(idx_hbm.at[wid], idx_v)

    def gather(u, b):
        return pltpu.make_async_copy(
            table_hbm.at[idx_v.at[u, pl.ds(0, CHUNK)]],
            rows_v.at[b], gsems.at[b])

    def scatter(u, b):
        return pltpu.make_async_copy(
            rows_v.at[b],
            out_hbm.at[pl.ds(base + u * CHUNK, CHUNK)],
            ssems.at[b])

    for b in range(NBUF):
        gather(b, b).start()

    def step(u, b):
        bp = (b - 1) % NBUF
        gather(u, b).wait()
        scatter(u, b).start()
        scatter(u - 1, bp).wait()
        nxt = u - 1 + NBUF

        @pl.when(nxt < NU)
        def _():
            gather(nxt, bp).start()

    gather(0, 0).wait()
    scatter(0, 0).start()

    def body(t, carry):
        for b in range(NBUF):
            step(t * NBUF + b + 1, (b + 1) % NBUF)
        return carry

    ngrp = (NU - 1) // NBUF
    lax.fori_loop(0, ngrp, body, 0)
    for i in range(NU - 1 - ngrp * NBUF):
        step(ngrp * NBUF + 1 + i, (i + 1) % NBUF)
    scatter(NU - 1, (NU - 1) % NBUF).wait()


@jax.jit
def _gather(idx_r, table):
    mesh = plsc.VectorSubcoreMesh(core_axis_name="c", subcore_axis_name="s")
    k = functools.partial(
        pl.kernel,
        mesh=mesh,
        out_type=jax.ShapeDtypeStruct((B, D), jnp.float32),
        scratch_types=[
            pltpu.VMEM((NU, 2 * CHUNK), jnp.int32),
            pltpu.VMEM((NBUF, CHUNK, D), jnp.float32),
            pltpu.SemaphoreType.DMA((NBUF,)),
            pltpu.SemaphoreType.DMA((NBUF,)),
        ],
    )(_gather_body)
    return k(idx_r, table)


def kernel(idx, table):
    idx4 = jnp.reshape(idx.astype(jnp.int32), (NW, NU, CHUNK))
    idx_r = jnp.concatenate([idx4, jnp.zeros_like(idx4)], axis=-1)
    return _gather(idx_r, table)
